# baseline (speedup 1.0000x reference)
import functools

import jax
import jax.numpy as jnp
from jax import lax
from jax.experimental import pallas as pl
from jax.experimental.pallas import tpu as pltpu
from jax.experimental.pallas import tpu_sc as plsc

B = 16384
D = 200
H1 = 128
NUM_EMB = 6

_info = plsc.get_sparse_core_info()
_NC, _NS = _info.num_cores, _info.num_subcores
_NW = _NC * _NS
_BPW = B // _NW
_CH = 128

_VB = 4000


def _proj_kernel(t1_ref, t2_ref, tm_ref, w_ref, *p_refs):
    srcs = (t1_ref, t2_ref, t2_ref, t2_ref, t2_ref, tm_ref)
    for j in range(NUM_EMB):
        p_refs[j][...] = lax.dot_general(
            srcs[j][...], w_ref[j * D:(j + 1) * D, :],
            (((1,), (0,)), ((), ())), preferred_element_type=jnp.float32)


def _gather_kernel(p0, p1, p2, p3, p4, p5, i0, i1, i2, i3, i4, i5, out,
                   idx_v, rows_v, sem):
    wid = lax.axis_index("s") * _NC + lax.axis_index("c")
    base = wid * _BPW
    tables = (p0, p1, p2, p3, p4, p5)
    idxs = (i0, i1, i2, i3, i4, i5)
    for j in range(NUM_EMB):
        pltpu.sync_copy(idxs[j].at[pl.ds(base, _BPW)], idx_v.at[j])
    for c in range(_BPW // _CH):
        copies = [
            pltpu.async_copy(
                tables[j].at[idx_v.at[j, pl.ds(c * _CH, _CH)]],
                rows_v.at[j], sem)
            for j in range(NUM_EMB)
        ]
        for cp in copies:
            cp.wait()

        def _sum_body(i, _):
            r = i // (H1 // 16)
            l = (i % (H1 // 16)) * 16
            v = (rows_v[0, r, pl.ds(l, 16)] + rows_v[1, r, pl.ds(l, 16)] +
                 rows_v[2, r, pl.ds(l, 16)] + rows_v[3, r, pl.ds(l, 16)] +
                 rows_v[4, r, pl.ds(l, 16)] + rows_v[5, r, pl.ds(l, 16)])
            rows_v[0, r, pl.ds(l, 16)] = v
            return _

        lax.fori_loop(0, _CH * (H1 // 16), _sum_body, None)
        pltpu.sync_copy(rows_v.at[0],
                        out.at[pl.ds(base + c * _CH, _CH)])


_gather = functools.partial(
    pl.kernel,
    mesh=plsc.VectorSubcoreMesh(core_axis_name="c", subcore_axis_name="s"),
    out_type=jax.ShapeDtypeStruct((B, H1), jnp.float32),
    scratch_types=[
        pltpu.VMEM((NUM_EMB, _BPW), jnp.int32),
        pltpu.VMEM((NUM_EMB, _CH, H1), jnp.float32),
        pltpu.SemaphoreType.DMA,
    ],
)(_gather_kernel)


_BLK = 1024


def _finish_kernel(x_ref, w2_ref, w3_ref, b1_ref, b2_ref, b3_ref, out_ref):
    h = jnp.maximum(x_ref[...] + b1_ref[...], 0.0)
    h = lax.dot_general(h, w2_ref[...], (((1,), (0,)), ((), ())),
                        preferred_element_type=jnp.float32)
    h = jnp.maximum(h + b2_ref[...], 0.0)
    logits = lax.dot_general(h, w3_ref[...], (((1,), (0,)), ((), ())),
                             preferred_element_type=jnp.float32)
    logits = logits + b3_ref[...]
    m = jnp.max(logits, axis=1, keepdims=True)
    e = jnp.exp(logits - m)
    out_ref[...] = e / jnp.sum(e, axis=1, keepdims=True)


def kernel(user_embedding1, user_embedding2, user_embedding3,
           user_embedding4, user_embedding5, movie_embedding,
           T_u1, T_u2, T_movie, W1, b1, W2, b2, W3, b3):
    idx = [jnp.asarray(i, jnp.int32) for i in
           (user_embedding1, user_embedding2, user_embedding3,
            user_embedding4, user_embedding5, movie_embedding)]
    w1t = W1.T

    vocab = T_u1.shape[0]
    proj = pl.pallas_call(
        _proj_kernel,
        grid=(vocab // _VB,),
        in_specs=[
            pl.BlockSpec((_VB, D), lambda i: (i, 0)),
            pl.BlockSpec((_VB, D), lambda i: (i, 0)),
            pl.BlockSpec((_VB, D), lambda i: (i, 0)),
            pl.BlockSpec((NUM_EMB * D, H1), lambda i: (0, 0)),
        ],
        out_specs=[pl.BlockSpec((_VB, H1), lambda i: (i, 0))
                   for _ in range(NUM_EMB)],
        out_shape=[jax.ShapeDtypeStruct((vocab, H1), jnp.float32)
                   for _ in range(NUM_EMB)],
    )(T_u1, T_u2, T_movie, w1t)

    xsum = _gather(*proj, *idx)

    grid = B // _BLK
    out = pl.pallas_call(
        _finish_kernel,
        grid=(grid,),
        in_specs=[
            pl.BlockSpec((_BLK, H1), lambda i: (i, 0)),
            pl.BlockSpec((H1, 64), lambda i: (0, 0)),
            pl.BlockSpec((64, 5), lambda i: (0, 0)),
            pl.BlockSpec((1, H1), lambda i: (0, 0)),
            pl.BlockSpec((1, 64), lambda i: (0, 0)),
            pl.BlockSpec((1, 5), lambda i: (0, 0)),
        ],
        out_specs=pl.BlockSpec((_BLK, 5), lambda i: (i, 0)),
        out_shape=jax.ShapeDtypeStruct((B, 5), jnp.float32),
    )(xsum, W2.T, W3.T, b1.reshape(1, -1), b2.reshape(1, -1),
      b3.reshape(1, -1))
    return out


# speedup vs baseline: 1.0278x; 1.0278x over previous
import functools

import jax
import jax.numpy as jnp
from jax import lax
from jax.experimental import pallas as pl
from jax.experimental.pallas import tpu as pltpu
from jax.experimental.pallas import tpu_sc as plsc

B = 16384
D = 200
H1 = 128
NUM_EMB = 6

_info = plsc.get_sparse_core_info()
_NC, _NS = _info.num_cores, _info.num_subcores
_NW = _NC * _NS
_BPW = B // _NW
_CH = 128

_VB = 4000


def _proj_kernel(t1_ref, t2_ref, tm_ref, w_ref, *p_refs):
    srcs = (t1_ref, t2_ref, t2_ref, t2_ref, t2_ref, tm_ref)
    for j in range(NUM_EMB):
        p_refs[j][...] = lax.dot_general(
            srcs[j][...], w_ref[j * D:(j + 1) * D, :],
            (((1,), (0,)), ((), ())), preferred_element_type=jnp.float32)


def _gather_kernel(p0, p1, p2, p3, p4, p5, i0, i1, i2, i3, i4, i5, out,
                   idx_v, rows_v, sem):
    wid = lax.axis_index("s") * _NC + lax.axis_index("c")
    base = wid * _BPW
    tables = (p0, p1, p2, p3, p4, p5)
    idxs = (i0, i1, i2, i3, i4, i5)
    nch = _BPW // _CH
    chunks = [(j, c) for j in range(NUM_EMB) for c in range(nch)]
    copies = []
    for j in range(NUM_EMB):
        pltpu.sync_copy(idxs[j].at[pl.ds(base, _BPW)], idx_v.at[j])
    for i, (j, c) in enumerate(chunks):
        buf = rows_v.at[i % 2]
        if i >= 2:
            copies[i - 2].wait()
            pj, pc = chunks[i - 2]
            pltpu.sync_copy(
                buf, out.at[pl.ds(pj * B + wid * _BPW + pc * _CH, _CH)])
        copies.append(pltpu.async_copy(
            tables[j].at[idx_v.at[j, pl.ds(c * _CH, _CH)]], buf, sem))
    for i in (len(chunks) - 2, len(chunks) - 1):
        copies[i].wait()
        j, c = chunks[i]
        pltpu.sync_copy(
            rows_v.at[i % 2],
            out.at[pl.ds(j * B + wid * _BPW + c * _CH, _CH)])


_gather = functools.partial(
    pl.kernel,
    mesh=plsc.VectorSubcoreMesh(core_axis_name="c", subcore_axis_name="s"),
    out_type=jax.ShapeDtypeStruct((NUM_EMB * B, H1), jnp.float32),
    scratch_types=[
        pltpu.VMEM((NUM_EMB, _BPW), jnp.int32),
        pltpu.VMEM((2, _CH, H1), jnp.float32),
        pltpu.SemaphoreType.DMA,
    ],
)(_gather_kernel)


_BLK = 1024


def _finish_kernel(x_ref, w2_ref, w3_ref, b1_ref, b2_ref, b3_ref, out_ref):
    h = x_ref[0] + x_ref[1] + x_ref[2] + x_ref[3] + x_ref[4] + x_ref[5]
    h = jnp.maximum(h + b1_ref[...], 0.0)
    h = lax.dot_general(h, w2_ref[...], (((1,), (0,)), ((), ())),
                        preferred_element_type=jnp.float32)
    h = jnp.maximum(h + b2_ref[...], 0.0)
    logits = lax.dot_general(h, w3_ref[...], (((1,), (0,)), ((), ())),
                             preferred_element_type=jnp.float32)
    logits = logits + b3_ref[...]
    m = jnp.max(logits, axis=1, keepdims=True)
    e = jnp.exp(logits - m)
    out_ref[...] = e / jnp.sum(e, axis=1, keepdims=True)


def kernel(user_embedding1, user_embedding2, user_embedding3,
           user_embedding4, user_embedding5, movie_embedding,
           T_u1, T_u2, T_movie, W1, b1, W2, b2, W3, b3):
    idx = [jnp.asarray(i, jnp.int32) for i in
           (user_embedding1, user_embedding2, user_embedding3,
            user_embedding4, user_embedding5, movie_embedding)]
    w1t = W1.T

    vocab = T_u1.shape[0]
    proj = pl.pallas_call(
        _proj_kernel,
        grid=(vocab // _VB,),
        in_specs=[
            pl.BlockSpec((_VB, D), lambda i: (i, 0)),
            pl.BlockSpec((_VB, D), lambda i: (i, 0)),
            pl.BlockSpec((_VB, D), lambda i: (i, 0)),
            pl.BlockSpec((NUM_EMB * D, H1), lambda i: (0, 0)),
        ],
        out_specs=[pl.BlockSpec((_VB, H1), lambda i: (i, 0))
                   for _ in range(NUM_EMB)],
        out_shape=[jax.ShapeDtypeStruct((vocab, H1), jnp.float32)
                   for _ in range(NUM_EMB)],
    )(T_u1, T_u2, T_movie, w1t)

    rows = _gather(*proj, *idx)
    xall = rows.reshape(NUM_EMB, B, H1)

    grid = B // _BLK
    out = pl.pallas_call(
        _finish_kernel,
        grid=(grid,),
        in_specs=[
            pl.BlockSpec((NUM_EMB, _BLK, H1), lambda i: (0, i, 0)),
            pl.BlockSpec((H1, 64), lambda i: (0, 0)),
            pl.BlockSpec((64, 5), lambda i: (0, 0)),
            pl.BlockSpec((1, H1), lambda i: (0, 0)),
            pl.BlockSpec((1, 64), lambda i: (0, 0)),
            pl.BlockSpec((1, 5), lambda i: (0, 0)),
        ],
        out_specs=pl.BlockSpec((_BLK, 5), lambda i: (i, 0)),
        out_shape=jax.ShapeDtypeStruct((B, 5), jnp.float32),
    )(xall, W2.T, W3.T, b1.reshape(1, -1), b2.reshape(1, -1),
      b3.reshape(1, -1))
    return out
